# baseline (speedup 1.0000x reference)
import functools

import jax
import jax.numpy as jnp
from jax import lax
from jax.experimental import pallas as pl
from jax.experimental.pallas import tpu as pltpu
from jax.experimental.pallas import tpu_sc as plsc

NUM_CORES = 2
NUM_SUBCORES = 16
NUM_WORKERS = NUM_CORES * NUM_SUBCORES
NB = 128
TS = 130


@functools.lru_cache(maxsize=None)
def _make_gather(B0, H, V, D):
    cols_per_w = B0 // NUM_WORKERS
    n_blocks = cols_per_w // NB
    mesh = plsc.VectorSubcoreMesh(
        core_axis_name="c",
        subcore_axis_name="s",
        num_cores=NUM_CORES,
        num_subcores=NUM_SUBCORES,
    )

    @functools.partial(
        pl.kernel,
        out_type=jax.ShapeDtypeStruct((H, D, B0), jnp.float32),
        mesh=mesh,
        scratch_types=[
            pltpu.VMEM((NB,), jnp.int32),
            pltpu.VMEM((NB,), jnp.int32),
            pltpu.VMEM((NB, D), jnp.float32),
            pltpu.VMEM((NB, D), jnp.float32),
            pltpu.VMEM((D, TS), jnp.float32),
            pltpu.VMEM((D, TS), jnp.float32),
        ]
        + [pltpu.SemaphoreType.DMA] * 6,
        compiler_params=pltpu.CompilerParams(
            use_tc_tiling_on_sc=False, needs_layout_passes=False
        ),
    )
    def gather_kernel(idx_hbm, table_hbm, out_hbm, i0, i1, g0, g1, t0, t1, *sems):
        i_v = (i0, i1)
        g_v = (g0, g1)
        t_v = (t0, t1)
        sem_i = sems[0:2]
        sem_g = sems[2:4]
        sem_o = sems[4:6]
        wid = lax.axis_index("s") * NUM_CORES + lax.axis_index("c")

        def idx_copy(b0, h, p):
            return pltpu.async_copy(
                idx_hbm.at[h, pl.ds(b0, NB)], i_v[p], sem_i[p]
            )

        def idx_wait(b0, h, p):
            pltpu.make_async_copy(
                idx_hbm.at[h, pl.ds(b0, NB)], i_v[p], sem_i[p]
            ).wait()

        def gather(p):
            return pltpu.async_copy(table_hbm.at[i_v[p]], g_v[p], sem_g[p])

        def gather_wait(p):
            pltpu.make_async_copy(
                table_hbm.at[i_v[p]], g_v[p], sem_g[p]
            ).wait()

        def wb(b0, h, p):
            return pltpu.async_copy(
                t_v[p].at[:, pl.ds(0, NB)],
                out_hbm.at[h, :, pl.ds(b0, NB)],
                sem_o[p],
            )

        def wb_wait(b0, h, p):
            pltpu.make_async_copy(
                t_v[p].at[:, pl.ds(0, NB)],
                out_hbm.at[h, :, pl.ds(b0, NB)],
                sem_o[p],
            ).wait()

        iotas = [
            lax.broadcasted_iota(jnp.int32, (16,), 0) + 16 * m
            for m in range(NB // 16)
        ]

        def transpose(p):
            def body(d, carry):
                cols = jnp.full((16,), d, dtype=jnp.int32)
                for m in range(NB // 16):
                    v = plsc.load_gather(g_v[p], [iotas[m], cols])
                    t_v[p][d, pl.ds(16 * m, 16)] = v
                return carry

            lax.fori_loop(0, D, body, 0)

        def block_body(bc, carry):
            b0 = wid * cols_per_w + bc * NB
            idx_copy(b0, 0, 0)
            idx_wait(b0, 0, 0)
            gather(0)
            idx_copy(b0, 1, 1)

            def h_body(ho, carry2):
                for hh in range(2):
                    h = ho * 2 + hh
                    p = hh
                    q = (hh + 1) % 2
                    gather_wait(p)

                    def fire_gather():
                        idx_wait(b0, h + 1, q)
                        gather(q)
                        return None

                    if hh == 0:
                        fire_gather()
                    else:
                        pl.when(ho < H // 2 - 1)(fire_gather)

                    def fire_idx():
                        idx_copy(b0, h + 2, p)
                        return None

                    pl.when(ho < H // 2 - 1)(fire_idx)

                    def buf_free():
                        wb_wait(b0, h - 2, p)

                    pl.when(ho >= 1)(buf_free)
                    transpose(p)
                    wb(b0, h, p)
                return carry2

            lax.fori_loop(0, H // 2, h_body, 0)
            wb_wait(b0, H - 2, 0)
            wb_wait(b0, H - 1, 1)
            return carry

        lax.fori_loop(0, n_blocks, block_body, 0)

    return gather_kernel


def kernel(indices, table):
    B0, H = indices.shape
    V, D = table.shape
    idx_t = jnp.transpose(indices).astype(jnp.int32)
    out_t = _make_gather(B0, H, V, D)(idx_t, table)
    return jnp.transpose(out_t, (2, 0, 1))


# speedup vs baseline: 2.1190x; 2.1190x over previous
import functools

import jax
import jax.numpy as jnp
from jax import lax
from jax.experimental import pallas as pl
from jax.experimental.pallas import tpu as pltpu
from jax.experimental.pallas import tpu_sc as plsc

NUM_CORES = 2
NUM_SUBCORES = 16
NUM_WORKERS = NUM_CORES * NUM_SUBCORES
NB = 128
TS = 129


@functools.lru_cache(maxsize=None)
def _make_gather(B0, H, V, D):
    cols_per_w = B0 // NUM_WORKERS
    n_blocks = cols_per_w // NB
    mesh = plsc.VectorSubcoreMesh(
        core_axis_name="c",
        subcore_axis_name="s",
        num_cores=NUM_CORES,
        num_subcores=NUM_SUBCORES,
    )

    @functools.partial(
        pl.kernel,
        out_type=jax.ShapeDtypeStruct((H, D, B0), jnp.float32),
        mesh=mesh,
        scratch_types=[
            pltpu.VMEM((NB,), jnp.int32),
            pltpu.VMEM((NB,), jnp.int32),
            pltpu.VMEM((NB, D), jnp.float32),
            pltpu.VMEM((NB, D), jnp.float32),
            pltpu.VMEM((D, TS), jnp.float32),
            pltpu.VMEM((D, TS), jnp.float32),
        ]
        + [pltpu.SemaphoreType.DMA] * 6,
        compiler_params=pltpu.CompilerParams(
            use_tc_tiling_on_sc=False, needs_layout_passes=False
        ),
    )
    def gather_kernel(idx_hbm, table_hbm, out_hbm, i0, i1, g0, g1, t0, t1, *sems):
        i_v = (i0, i1)
        g_v = (g0, g1)
        t_v = (t0, t1)
        sem_i = sems[0:2]
        sem_g = sems[2:4]
        sem_o = sems[4:6]
        wid = lax.axis_index("s") * NUM_CORES + lax.axis_index("c")

        def idx_copy(b0, h, p):
            return pltpu.async_copy(
                idx_hbm.at[h, pl.ds(b0, NB)], i_v[p], sem_i[p]
            )

        def idx_wait(b0, h, p):
            pltpu.make_async_copy(
                idx_hbm.at[h, pl.ds(b0, NB)], i_v[p], sem_i[p]
            ).wait()

        def gather(p):
            return pltpu.async_copy(table_hbm.at[i_v[p]], g_v[p], sem_g[p])

        def gather_wait(p):
            pltpu.make_async_copy(
                table_hbm.at[i_v[p]], g_v[p], sem_g[p]
            ).wait()

        def wb(b0, h, p):
            return pltpu.async_copy(
                t_v[p].at[:, pl.ds(0, NB)],
                out_hbm.at[h, :, pl.ds(b0, NB)],
                sem_o[p],
            )

        def wb_wait(b0, h, p):
            pltpu.make_async_copy(
                t_v[p].at[:, pl.ds(0, NB)],
                out_hbm.at[h, :, pl.ds(b0, NB)],
                sem_o[p],
            ).wait()

        iotas = [
            lax.broadcasted_iota(jnp.int32, (16,), 0) + 16 * k
            for k in range(D // 16)
        ]

        def transpose(p):
            def body(r4, carry):
                for rr in range(4):
                    r = r4 * 4 + rr
                    cols = jnp.full((16,), r, dtype=jnp.int32)
                    for k in range(D // 16):
                        v = g_v[p][r, pl.ds(16 * k, 16)]
                        plsc.store_scatter(t_v[p], [iotas[k], cols], v)
                return carry

            lax.fori_loop(0, NB // 4, body, 0)

        def block_body(bc, carry):
            b0 = wid * cols_per_w + bc * NB
            idx_copy(b0, 0, 0)
            idx_wait(b0, 0, 0)
            gather(0)
            idx_copy(b0, 1, 1)

            def h_body(ho, carry2):
                for hh in range(2):
                    h = ho * 2 + hh
                    p = hh
                    q = (hh + 1) % 2
                    gather_wait(p)

                    def fire_gather():
                        idx_wait(b0, h + 1, q)
                        gather(q)
                        return None

                    if hh == 0:
                        fire_gather()
                    else:
                        pl.when(ho < H // 2 - 1)(fire_gather)

                    def fire_idx():
                        idx_copy(b0, h + 2, p)
                        return None

                    pl.when(ho < H // 2 - 1)(fire_idx)

                    def buf_free():
                        wb_wait(b0, h - 2, p)

                    pl.when(ho >= 1)(buf_free)
                    transpose(p)
                    wb(b0, h, p)
                return carry2

            lax.fori_loop(0, H // 2, h_body, 0)
            wb_wait(b0, H - 2, 0)
            wb_wait(b0, H - 1, 1)
            return carry

        lax.fori_loop(0, n_blocks, block_body, 0)

    return gather_kernel


def kernel(indices, table):
    B0, H = indices.shape
    V, D = table.shape
    idx_t = jnp.transpose(indices).astype(jnp.int32)
    out_t = _make_gather(B0, H, V, D)(idx_t, table)
    return jnp.transpose(out_t, (2, 0, 1))


# speedup vs baseline: 2.7357x; 1.2910x over previous
import functools

import jax
import jax.numpy as jnp
from jax import lax
from jax.experimental import pallas as pl
from jax.experimental.pallas import tpu as pltpu
from jax.experimental.pallas import tpu_sc as plsc

NUM_CORES = 2
NUM_SUBCORES = 16
NUM_WORKERS = NUM_CORES * NUM_SUBCORES
NB = 128
TS = 129


@functools.lru_cache(maxsize=None)
def _make_gather(B0, H, V, D):
    cols_per_w = B0 // NUM_WORKERS
    n_blocks = cols_per_w // NB
    mesh = plsc.VectorSubcoreMesh(
        core_axis_name="c",
        subcore_axis_name="s",
        num_cores=NUM_CORES,
        num_subcores=NUM_SUBCORES,
    )

    @functools.partial(
        pl.kernel,
        out_type=jax.ShapeDtypeStruct((H, D, B0), jnp.float32),
        mesh=mesh,
        scratch_types=[
            pltpu.VMEM((NB,), jnp.int32),
            pltpu.VMEM((NB,), jnp.int32),
            pltpu.VMEM((NB, D), jnp.float32),
            pltpu.VMEM((NB, D), jnp.float32),
            pltpu.VMEM((D, TS), jnp.float32),
            pltpu.VMEM((D, TS), jnp.float32),
        ]
        + [pltpu.SemaphoreType.DMA] * 6,
        compiler_params=pltpu.CompilerParams(
            use_tc_tiling_on_sc=False, needs_layout_passes=False
        ),
    )
    def gather_kernel(idx_hbm, table_hbm, out_hbm, i0, i1, g0, g1, t0, t1, *sems):
        i_v = (i0, i1)
        g_v = (g0, g1)
        t_v = (t0, t1)
        sem_i = sems[0:2]
        sem_g = sems[2:4]
        sem_o = sems[4:6]
        wid = lax.axis_index("s") * NUM_CORES + lax.axis_index("c")

        def idx_copy(b0, h, p):
            return pltpu.async_copy(
                idx_hbm.at[h, pl.ds(b0, NB)], i_v[p], sem_i[p]
            )

        def idx_wait(b0, h, p):
            pltpu.make_async_copy(
                idx_hbm.at[h, pl.ds(b0, NB)], i_v[p], sem_i[p]
            ).wait()

        def gather(p):
            return pltpu.async_copy(table_hbm.at[i_v[p]], g_v[p], sem_g[p])

        def gather_wait(p):
            pltpu.make_async_copy(
                table_hbm.at[i_v[p]], g_v[p], sem_g[p]
            ).wait()

        def wb(b0, h, p):
            return pltpu.async_copy(
                t_v[p].at[:, pl.ds(0, NB)],
                out_hbm.at[h, :, pl.ds(b0, NB)],
                sem_o[p],
            )

        def wb_wait(b0, h, p):
            pltpu.make_async_copy(
                t_v[p].at[:, pl.ds(0, NB)],
                out_hbm.at[h, :, pl.ds(b0, NB)],
                sem_o[p],
            ).wait()

        iotas = [
            lax.broadcasted_iota(jnp.int32, (16,), 0) + 16 * k
            for k in range(D // 16)
        ]

        def transpose(p):
            @plsc.parallel_loop(0, NB, unroll=8)
            def _(r):
                cols = jnp.full((16,), r, dtype=jnp.int32)
                for k in range(D // 16):
                    v = g_v[p][r, pl.ds(16 * k, 16)]
                    plsc.store_scatter(t_v[p], [iotas[k], cols], v)

        def block_body(bc, carry):
            b0 = wid * cols_per_w + bc * NB
            idx_copy(b0, 0, 0)
            idx_wait(b0, 0, 0)
            gather(0)
            idx_copy(b0, 1, 1)

            def h_body(ho, carry2):
                for hh in range(2):
                    h = ho * 2 + hh
                    p = hh
                    q = (hh + 1) % 2
                    gather_wait(p)

                    def fire_gather():
                        idx_wait(b0, h + 1, q)
                        gather(q)
                        return None

                    if hh == 0:
                        fire_gather()
                    else:
                        pl.when(ho < H // 2 - 1)(fire_gather)

                    def fire_idx():
                        idx_copy(b0, h + 2, p)
                        return None

                    pl.when(ho < H // 2 - 1)(fire_idx)

                    def buf_free():
                        wb_wait(b0, h - 2, p)

                    pl.when(ho >= 1)(buf_free)
                    transpose(p)
                    wb(b0, h, p)
                return carry2

            lax.fori_loop(0, H // 2, h_body, 0)
            wb_wait(b0, H - 2, 0)
            wb_wait(b0, H - 1, 1)
            return carry

        lax.fori_loop(0, n_blocks, block_body, 0)

    return gather_kernel


def kernel(indices, table):
    B0, H = indices.shape
    V, D = table.shape
    idx_t = jnp.transpose(indices).astype(jnp.int32)
    out_t = _make_gather(B0, H, V, D)(idx_t, table)
    return jnp.transpose(out_t, (2, 0, 1))
